# Initial kernel scaffold; baseline (speedup 1.0000x reference)
import jax
import jax.numpy as jnp
from jax.experimental import pallas as pl


def kernel(x, edge_index, edge_weight, W0_rel, b0_rel, W0_root, W1_rel, b1_rel, W1_root, W_skip, a0, a1):
    raise NotImplementedError("write your pallas kernel here")


# speedup vs baseline: 4.8263x; 4.8263x over previous
import functools

import jax
import jax.numpy as jnp
from jax import lax
from jax.experimental import pallas as pl
from jax.experimental.pallas import tpu as pltpu
from jax.experimental.pallas import tpu_sc as plsc

NC = 2
NS = 16
NW = NC * NS
L = 16
CHUNK = 128

F32 = jnp.float32
I32 = jnp.int32


def _build_sc_pass(n_pad, ch, d, weighted):
    mesh = plsc.VectorSubcoreMesh(
        core_axis_name="c", subcore_axis_name="s", num_cores=NC,
        num_subcores=NS)
    r_tile = n_pad // NS
    zc = r_tile // CHUNK

    out_type = [jax.ShapeDtypeStruct((NC, n_pad, d), F32)]
    if not weighted:
        out_type.append(jax.ShapeDtypeStruct((NC, n_pad), F32))

    scratch = [
        pltpu.VMEM((ch, CHUNK), I32),
        pltpu.VMEM((ch, CHUNK), I32),
        pltpu.VMEM((CHUNK, d), F32),
        pltpu.VMEM_SHARED((n_pad, d), F32),
    ]
    if weighted:
        scratch.append(pltpu.VMEM((CHUNK,), F32))
    else:
        scratch.append(pltpu.VMEM_SHARED((n_pad,), F32))
        scratch.append(pltpu.VMEM((CHUNK,), F32))
    scratch.append(pltpu.SemaphoreType.DMA)

    def body(*refs):
        if weighted:
            (tab_hbm, src_hbm, dst_hbm, w_hbm, out_hbm,
             src_v, dst_v, rows_v, acc_sh, w_v, sem) = refs
        else:
            (tab_hbm, src_hbm, dst_hbm, out_hbm, cnt_hbm,
             src_v, dst_v, rows_v, acc_sh, cnt_sh, ones_v,
             sem) = refs
        cid = lax.axis_index("c")
        sid = lax.axis_index("s")
        wid = sid * NC + cid
        base = sid * r_tile

        def zrow(i, _):
            for g in range(d // L):
                rows_v[i, pl.ds(g * L, L)] = jnp.zeros((L,), F32)
            return 0
        lax.fori_loop(0, CHUNK, zrow, 0)
        if not weighted:
            for g in range(CHUNK // L):
                ones_v[pl.ds(g * L, L)] = jnp.ones((L,), F32)

        for k in range(zc):
            pltpu.sync_copy(rows_v, acc_sh.at[pl.ds(base + k * CHUNK, CHUNK)])
            if not weighted:
                pltpu.sync_copy(rows_v.at[0],
                                cnt_sh.at[pl.ds(base + k * CHUNK, CHUNK)])
        plsc.subcore_barrier()

        pltpu.sync_copy(src_hbm.at[wid], src_v)
        pltpu.sync_copy(dst_hbm.at[wid], dst_v)

        def edge_chunk(j, _):
            pltpu.async_copy(tab_hbm.at[src_v.at[j]], rows_v, sem).wait()
            if weighted:
                pltpu.sync_copy(w_hbm.at[wid, j], w_v)
                for gg in range(CHUNK // L):
                    wv16 = w_v[pl.ds(gg * L, L)]
                    for lane in range(L):
                        e = gg * L + lane
                        wb = jnp.full((L,), 1.0, F32) * wv16[lane]
                        for g in range(d // L):
                            rows_v[e, pl.ds(g * L, L)] = (
                                rows_v[e, pl.ds(g * L, L)] * wb)
            pltpu.sync_copy(rows_v, acc_sh.at[dst_v.at[j]], add=True)
            if not weighted:
                pltpu.sync_copy(ones_v, cnt_sh.at[dst_v.at[j]], add=True)
            return 0
        lax.fori_loop(0, ch, edge_chunk, 0)
        plsc.subcore_barrier()

        pltpu.sync_copy(acc_sh.at[pl.ds(base, r_tile)],
                        out_hbm.at[cid, pl.ds(base, r_tile)])
        if not weighted:
            pltpu.sync_copy(cnt_sh.at[pl.ds(base, r_tile)],
                            cnt_hbm.at[cid, pl.ds(base, r_tile)])

    return pl.kernel(body, out_type=tuple(out_type), mesh=mesh,
                     scratch_types=scratch)


def _dot_t(a, w):
    return lax.dot_general(a, w, (((1,), (1,)), ((), ())),
                           preferred_element_type=F32)


def _dense0_body(agg_ref, cnt_ref, x_ref, w0rel_ref, b0_ref, w0root_ref,
                 wskip_ref, a0_ref, out_ref):
    cnt = jnp.maximum(cnt_ref[0] + cnt_ref[1], 1.0)
    agg = (agg_ref[0] + agg_ref[1]) / cnt
    x = x_ref[...]
    h = _dot_t(agg, w0rel_ref[...]) + b0_ref[...] + _dot_t(x, w0root_ref[...])
    a = a0_ref[...]
    h = jnp.where(h >= 0, h, a * h)
    h = jnp.where(h >= 0, h, a * h)
    out_ref[...] = h + _dot_t(x, wskip_ref[...])


def _dense1_body(agg_ref, cnt_ref, z_ref, w1rel_ref, b1_ref, w1root_ref,
                 a1_ref, out_ref):
    cnt = jnp.maximum(cnt_ref[0] + cnt_ref[1], 1.0)
    agg = (agg_ref[0] + agg_ref[1]) / cnt
    z = z_ref[...]
    h = _dot_t(agg, w1rel_ref[...]) + b1_ref[...] + _dot_t(z, w1root_ref[...])
    out_ref[...] = jnp.where(h >= 0, h, a1_ref[...] * h)


def _dense_call(body, n_pad, d, n_in_mats):
    BT = 1024
    grid = (n_pad // BT,)
    parts = pl.BlockSpec((NC, BT, d), lambda i: (0, i, 0))
    cnt = pl.BlockSpec((NC, BT, 1), lambda i: (0, i, 0))
    rows = pl.BlockSpec((BT, d), lambda i: (i, 0))
    mat = pl.BlockSpec((d, d), lambda i: (0, 0))
    vec = pl.BlockSpec((1, d), lambda i: (0, 0))
    if n_in_mats == 3:
        in_specs = [parts, cnt, rows, mat, vec, mat, mat, vec]
    else:
        in_specs = [parts, cnt, rows, mat, vec, mat, vec]
    return pl.pallas_call(
        body, grid=grid, in_specs=in_specs, out_specs=rows,
        out_shape=jax.ShapeDtypeStruct((n_pad, d), F32))


def kernel(x, edge_index, edge_weight, W0_rel, b0_rel, W0_root,
           W1_rel, b1_rel, W1_root, W_skip, a0, a1):
    N, D = x.shape
    E = edge_index.shape[1]
    ch = -(-E // (NW * CHUNK))
    e_pad = NW * ch * CHUNK
    n_pad = -(-(N + 1) // (NS * CHUNK)) * (NS * CHUNK)

    pad_e = e_pad - E
    src_r = jnp.concatenate(
        [edge_index[0], jnp.zeros((pad_e,), I32)]).reshape(NW, ch, CHUNK)
    dst_r = jnp.concatenate(
        [edge_index[1], jnp.full((pad_e,), N, I32)]).reshape(NW, ch, CHUNK)
    w_r = jnp.concatenate(
        [edge_weight, jnp.zeros((pad_e,), F32)]).reshape(NW, ch, CHUNK)
    x_pad = jnp.pad(x, ((0, n_pad - N), (0, 0)))

    sc0 = _build_sc_pass(n_pad, ch, D, weighted=False)
    sc1 = _build_sc_pass(n_pad, ch, D, weighted=True)

    agg0_parts, cnt_parts = sc0(x_pad, src_r, dst_r)
    cnt_r = cnt_parts[..., None]

    z2_pad = _dense_call(_dense0_body, n_pad, D, 3)(
        agg0_parts, cnt_r, x_pad, W0_rel, b0_rel.reshape(1, D), W0_root,
        W_skip, a0.reshape(1, D))

    (agg1_parts,) = sc1(z2_pad, src_r, dst_r, w_r)

    out_pad = _dense_call(_dense1_body, n_pad, D, 2)(
        agg1_parts, cnt_r, z2_pad, W1_rel, b1_rel.reshape(1, D), W1_root,
        a1.reshape(1, D))
    return out_pad[:N]
